# Initial kernel scaffold; baseline (speedup 1.0000x reference)
import jax
import jax.numpy as jnp
from jax.experimental import pallas as pl


def kernel(x, edge_index, W1, b1, W2, b2, Wl, bl):
    raise NotImplementedError("write your pallas kernel here")


# speedup vs baseline: 15.0583x; 15.0583x over previous
import functools

import jax
import jax.numpy as jnp
from jax import lax
from jax.experimental import pallas as pl
from jax.experimental.pallas import tpu as pltpu
from jax.experimental.pallas import tpu_sc as plsc

F32 = jnp.float32
I32 = jnp.int32
L = 16
NSUB = 16
NCORE = 2
S = 8


def kernel(x, edge_index, W1, b1, W2, b2, Wl, bl):
    N, IN_DIM = x.shape
    E = edge_index.shape[1]
    H = W1.shape[1]

    NPAD = ((N + 1 + 2047) // 2048) * 2048
    ROWS_T = NPAD // NSUB
    EPAD = ((E + 32767) // 32768) * 32768
    EROWS = EPAD // 128
    RW_SPLIT = EROWS // (NSUB * NCORE)
    RW_FULL = EROWS // NSUB
    NB_SPLIT = RW_SPLIT // S
    NB_FULL = RW_FULL // S
    D1 = 8
    C2 = H // L
    CPC = C2 // NCORE

    src = edge_index[0].astype(I32)
    dst = edge_index[1].astype(I32)
    pad = EPAD - E
    srcp = jnp.concatenate([src, jnp.zeros((pad,), I32)]).reshape(EROWS, 128)
    dstp = jnp.concatenate([dst, jnp.full((pad,), N, I32)]).reshape(EROWS, 128)

    mesh = plsc.VectorSubcoreMesh(core_axis_name="c", subcore_axis_name="s",
                                  num_cores=NCORE, num_subcores=NSUB)
    sc_params = pltpu.CompilerParams(use_tc_tiling_on_sc=False)

    zeros1 = jnp.zeros((NPAD,), F32)
    zeros_d1 = jnp.zeros((NPAD, D1), F32)
    zeros_d2 = jnp.zeros((NPAD, L), F32)

    @functools.partial(
        pl.kernel, mesh=mesh,
        out_type=jax.ShapeDtypeStruct((NCORE, NPAD), F32),
        compiler_params=sc_params,
        scratch_types=[
            pltpu.VMEM((S, 128), I32),
            pltpu.VMEM((128,), F32),
            pltpu.VMEM_SHARED((NPAD,), F32),
        ],
    )
    def deg_kernel(dst_hbm, z_hbm, out_hbm, idx_v, ones_v, acc_sh):
        c = lax.axis_index("c")
        s = lax.axis_index("s")
        pltpu.sync_copy(z_hbm.at[pl.ds(s * ROWS_T, ROWS_T)],
                        acc_sh.at[pl.ds(s * ROWS_T, ROWS_T)])
        for i in range(128 // L):
            ones_v[pl.ds(i * L, L)] = jnp.ones((L,), F32)
        plsc.subcore_barrier()
        row0 = c * (NSUB * RW_SPLIT) + s * RW_SPLIT

        def body(b, carry):
            pltpu.sync_copy(dst_hbm.at[pl.ds(row0 + b * S, S)], idx_v)
            for j in range(S):
                pltpu.sync_copy(ones_v, acc_sh.at[idx_v.at[j]], add=True)
            return carry

        lax.fori_loop(0, NB_SPLIT, body, 0)
        plsc.subcore_barrier()
        pltpu.sync_copy(acc_sh.at[pl.ds(s * ROWS_T, ROWS_T)],
                        out_hbm.at[c, pl.ds(s * ROWS_T, ROWS_T)])

    deg2 = deg_kernel(dstp, zeros1)

    xp = jnp.pad(x, ((0, NPAD - N), (0, D1 - IN_DIM)))

    def tc1_body(deg2_ref, xp_ref, dinv_ref, xs1_ref):
        deg = deg2_ref[0, :] + deg2_ref[1, :] + 1.0
        dinv = lax.rsqrt(deg)
        dinv_ref[...] = dinv
        xs1_ref[...] = xp_ref[...] * dinv[:, None]

    R1 = 2048
    dinv, xs1 = pl.pallas_call(
        tc1_body,
        grid=(NPAD // R1,),
        in_specs=[pl.BlockSpec((NCORE, R1), lambda i: (0, i)),
                  pl.BlockSpec((R1, D1), lambda i: (i, 0))],
        out_specs=(pl.BlockSpec((R1,), lambda i: (i,)),
                   pl.BlockSpec((R1, D1), lambda i: (i, 0))),
        out_shape=(jax.ShapeDtypeStruct((NPAD,), F32),
                   jax.ShapeDtypeStruct((NPAD, D1), F32)),
    )(deg2, xp)

    @functools.partial(
        pl.kernel, mesh=mesh,
        out_type=jax.ShapeDtypeStruct((NCORE, NPAD, D1), F32),
        compiler_params=sc_params,
        scratch_types=[
            pltpu.VMEM((S, 128), I32),
            pltpu.VMEM((S, 128), I32),
            pltpu.VMEM((128, D1), F32),
            pltpu.VMEM_SHARED((NPAD, D1), F32),
            pltpu.SemaphoreType.DMA,
        ],
    )
    def spmm1_kernel(tbl_hbm, src_hbm, dst_hbm, z_hbm, out_hbm,
                     si_v, di_v, rows_v, acc_sh, sem):
        c = lax.axis_index("c")
        s = lax.axis_index("s")
        pltpu.sync_copy(z_hbm.at[pl.ds(s * ROWS_T, ROWS_T)],
                        acc_sh.at[pl.ds(s * ROWS_T, ROWS_T)])
        plsc.subcore_barrier()
        row0 = c * (NSUB * RW_SPLIT) + s * RW_SPLIT

        def body(b, carry):
            pltpu.sync_copy(src_hbm.at[pl.ds(row0 + b * S, S)], si_v)
            pltpu.sync_copy(dst_hbm.at[pl.ds(row0 + b * S, S)], di_v)
            for j in range(S):
                pltpu.async_copy(tbl_hbm.at[si_v.at[j]], rows_v, sem).wait()
                pltpu.sync_copy(rows_v, acc_sh.at[di_v.at[j]], add=True)
            return carry

        lax.fori_loop(0, NB_SPLIT, body, 0)
        plsc.subcore_barrier()
        pltpu.sync_copy(acc_sh.at[pl.ds(s * ROWS_T, ROWS_T)],
                        out_hbm.at[c, pl.ds(s * ROWS_T, ROWS_T)])

    t1 = spmm1_kernel(xs1, srcp, dstp, zeros_d1)

    R = 2048

    def tc2_body(t1_ref, xs1_ref, dinv_ref, W1_ref, b1_ref, out_ref):
        dinv = dinv_ref[...]
        z1 = (t1_ref[0] + t1_ref[1] + xs1_ref[...]) * dinv[:, None]
        h1 = jnp.maximum(
            jnp.dot(z1[:, :IN_DIM], W1_ref[...], preferred_element_type=F32)
            + b1_ref[...][None, :], 0.0)
        xs2 = h1 * dinv[:, None]
        for k in range(C2):
            out_ref[k] = xs2[:, k * L:(k + 1) * L]

    xs2c = pl.pallas_call(
        tc2_body,
        grid=(NPAD // R,),
        in_specs=[pl.BlockSpec((NCORE, R, D1), lambda i: (0, i, 0)),
                  pl.BlockSpec((R, D1), lambda i: (i, 0)),
                  pl.BlockSpec((R,), lambda i: (i,)),
                  pl.BlockSpec((IN_DIM, H), lambda i: (0, 0)),
                  pl.BlockSpec((H,), lambda i: (0,))],
        out_specs=pl.BlockSpec((C2, R, L), lambda i: (0, i, 0)),
        out_shape=jax.ShapeDtypeStruct((C2, NPAD, L), F32),
    )(t1, xs1, dinv, W1, b1)

    @functools.partial(
        pl.kernel, mesh=mesh,
        out_type=jax.ShapeDtypeStruct((C2, NPAD, L), F32),
        compiler_params=sc_params,
        scratch_types=[
            pltpu.VMEM((S, 128), I32),
            pltpu.VMEM((S, 128), I32),
            pltpu.VMEM((S, 128), I32),
            pltpu.VMEM((128, L), F32),
            pltpu.VMEM_SHARED((NPAD, L), F32),
            pltpu.SemaphoreType.DMA,
        ],
    )
    def spmm2_kernel(tbl_hbm, src_hbm, dst_hbm, z_hbm, out_hbm,
                     si_v, di_v, gi_v, rows_v, acc_sh, sem):
        c = lax.axis_index("c")
        s = lax.axis_index("s")
        row0 = s * RW_FULL
        for k in range(CPC):
            chunk = c * CPC + k
            off = chunk * NPAD
            pltpu.sync_copy(z_hbm.at[pl.ds(s * ROWS_T, ROWS_T)],
                            acc_sh.at[pl.ds(s * ROWS_T, ROWS_T)])
            plsc.subcore_barrier()

            def body(b, carry):
                pltpu.sync_copy(src_hbm.at[pl.ds(row0 + b * S, S)], si_v)
                pltpu.sync_copy(dst_hbm.at[pl.ds(row0 + b * S, S)], di_v)
                for j in range(S):
                    for q in range(128 // L):
                        gi_v[j, pl.ds(q * L, L)] = (
                            si_v[j, pl.ds(q * L, L)] + off)
                    pltpu.async_copy(tbl_hbm.at[gi_v.at[j]], rows_v, sem).wait()
                    pltpu.sync_copy(rows_v, acc_sh.at[di_v.at[j]], add=True)
                return carry

            lax.fori_loop(0, NB_FULL, body, 0)
            plsc.subcore_barrier()
            pltpu.sync_copy(acc_sh.at[pl.ds(s * ROWS_T, ROWS_T)],
                            out_hbm.at[chunk, pl.ds(s * ROWS_T, ROWS_T)])

    t2 = spmm2_kernel(xs2c.reshape(C2 * NPAD, L), srcp, dstp, zeros_d2)

    def tc3_body(t2_ref, xs2_ref, dinv_ref, W2_ref, b2_ref, Wl_ref, bl_ref,
                 out_ref):
        dinv = dinv_ref[...]
        z2 = jnp.concatenate(
            [t2_ref[k] + xs2_ref[k] for k in range(C2)], axis=1) * dinv[:, None]
        h2 = jnp.maximum(
            jnp.dot(z2, W2_ref[...], preferred_element_type=F32)
            + b2_ref[...][None, :], 0.0)
        lg = jnp.dot(h2, Wl_ref[...], preferred_element_type=F32)[:, 0]
        out_ref[...] = lg + bl_ref[0]

    logits_pad = pl.pallas_call(
        tc3_body,
        grid=(NPAD // R,),
        in_specs=[pl.BlockSpec((C2, R, L), lambda i: (0, i, 0)),
                  pl.BlockSpec((C2, R, L), lambda i: (0, i, 0)),
                  pl.BlockSpec((R,), lambda i: (i,)),
                  pl.BlockSpec((H, H), lambda i: (0, 0)),
                  pl.BlockSpec((H,), lambda i: (0,)),
                  pl.BlockSpec((H, 1), lambda i: (0, 0)),
                  pl.BlockSpec((1,), lambda i: (0,))],
        out_specs=pl.BlockSpec((R,), lambda i: (i,)),
        out_shape=jax.ShapeDtypeStruct((NPAD,), F32),
    )(t2, xs2c, dinv, W2, b2, Wl, bl)

    return logits_pad[:N]


# speedup vs baseline: 28.2565x; 1.8765x over previous
import functools

import jax
import jax.numpy as jnp
from jax import lax
from jax.experimental import pallas as pl
from jax.experimental.pallas import tpu as pltpu
from jax.experimental.pallas import tpu_sc as plsc

F32 = jnp.float32
I32 = jnp.int32
L = 16
NSUB = 16
NCORE = 2
S1 = 10
S2 = 4


def kernel(x, edge_index, W1, b1, W2, b2, Wl, bl):
    N, IN_DIM = x.shape
    E = edge_index.shape[1]
    H = W1.shape[1]

    NPAD = ((N + 1 + 2047) // 2048) * 2048
    ROWS_T = NPAD // NSUB
    EBLK = 640 * 128
    EPAD = ((E + EBLK - 1) // EBLK) * EBLK
    EROWS = EPAD // 128
    RW_SPLIT = EROWS // (NSUB * NCORE)
    RW_FULL = EROWS // NSUB
    NB_SPLIT = RW_SPLIT // S1
    NB_FULL = RW_FULL // S2
    D1 = 8
    C2 = H // L
    CPC = C2 // NCORE

    src = edge_index[0].astype(I32)
    dst = edge_index[1].astype(I32)
    pad = EPAD - E
    fill = jnp.arange(pad, dtype=I32)
    srcp = jnp.concatenate([src, fill % 1024]).reshape(EROWS, 128)
    dstp = jnp.concatenate([dst, N + (fill % (NPAD - N))]).reshape(EROWS, 128)

    mesh = plsc.VectorSubcoreMesh(core_axis_name="c", subcore_axis_name="s",
                                  num_cores=NCORE, num_subcores=NSUB)
    sc_params = pltpu.CompilerParams(use_tc_tiling_on_sc=False)

    zeros1 = jnp.zeros((NPAD,), F32)
    zeros_d1 = jnp.zeros((NPAD, D1), F32)
    zeros_d2 = jnp.zeros((NPAD, L), F32)

    @functools.partial(
        pl.kernel, mesh=mesh,
        out_type=jax.ShapeDtypeStruct((NCORE, NPAD), F32),
        compiler_params=sc_params,
        scratch_types=[
            pltpu.VMEM((2, S1, 128), I32),
            pltpu.VMEM((128,), F32),
            pltpu.VMEM_SHARED((NPAD,), F32),
            pltpu.SemaphoreType.DMA,
            pltpu.SemaphoreType.DMA,
        ],
    )
    def deg_kernel(dst_hbm, z_hbm, out_hbm, di_v, ones_v, acc_sh, sm0, sm1):
        c = lax.axis_index("c")
        s = lax.axis_index("s")
        sems = (sm0, sm1)
        pltpu.sync_copy(z_hbm.at[pl.ds(s * ROWS_T, ROWS_T)],
                        acc_sh.at[pl.ds(s * ROWS_T, ROWS_T)])
        for i in range(128 // L):
            ones_v[pl.ds(i * L, L)] = jnp.ones((L,), F32)
        plsc.subcore_barrier()
        row0 = c * (NSUB * RW_SPLIT) + s * RW_SPLIT

        def stage(p, b):
            pltpu.sync_copy(dst_hbm.at[pl.ds(row0 + b * S1, S1)], di_v.at[p])
            for j in range(S1):
                pltpu.async_copy(ones_v, acc_sh.at[di_v.at[p, j]], sems[p],
                                 add=True)

        def drain(p):
            pltpu.make_async_copy(dst_hbm.at[pl.ds(0, S1)], di_v.at[p],
                                  sems[p]).wait()

        stage(0, 0)
        NB2 = NB_SPLIT // 2

        def body(i, carry):
            @pl.when(i > 0)
            def _():
                drain(1)

            stage(1, 2 * i + 1)

            @pl.when(i + 1 < NB2)
            def _():
                drain(0)
                stage(0, 2 * i + 2)

            return carry

        lax.fori_loop(0, NB2, body, 0)
        drain(0)
        drain(1)
        plsc.subcore_barrier()
        pltpu.sync_copy(acc_sh.at[pl.ds(s * ROWS_T, ROWS_T)],
                        out_hbm.at[c, pl.ds(s * ROWS_T, ROWS_T)])

    deg2 = deg_kernel(dstp, zeros1)

    def make_spmm(D, n_out, split, S):
        NB = NB_SPLIT if split else NB_FULL
        cpc = 1 if split else CPC
        SB = S * 128

        @functools.partial(
            pl.kernel, mesh=mesh,
            out_type=jax.ShapeDtypeStruct((n_out, NPAD, D), F32),
            compiler_params=sc_params,
            scratch_types=[
                pltpu.VMEM((2, S, 128), I32),
                pltpu.VMEM((2, S, 128), I32),
                pltpu.VMEM((2, S, 128), I32),
                pltpu.VMEM((2, SB, D), F32),
                pltpu.VMEM_SHARED((NPAD, D), F32),
                pltpu.SemaphoreType.DMA,
                pltpu.SemaphoreType.DMA,
                pltpu.SemaphoreType.DMA,
                pltpu.SemaphoreType.DMA,
            ],
        )
        def spmm_k(tbl_hbm, src_hbm, dst_hbm, z_hbm, out_hbm,
                   si_v, di_v, gi_v, rows_v, acc_sh, gs0, gs1, ss0, ss1):
            c = lax.axis_index("c")
            s = lax.axis_index("s")
            gsems = (gs0, gs1)
            ssems = (ss0, ss1)
            row0 = c * (NSUB * RW_SPLIT) + s * RW_SPLIT if split \
                else s * RW_FULL

            def prepare(p, b, off):
                pltpu.sync_copy(src_hbm.at[pl.ds(row0 + b * S, S)],
                                si_v.at[p])
                pltpu.sync_copy(dst_hbm.at[pl.ds(row0 + b * S, S)],
                                di_v.at[p])
                if split:
                    for j in range(S):
                        pltpu.async_copy(tbl_hbm.at[si_v.at[p, j]],
                                         rows_v.at[p, pl.ds(j * 128, 128)],
                                         gsems[p])
                else:
                    for j in range(S):
                        for q in range(128 // L):
                            gi_v[p, j, pl.ds(q * L, L)] = (
                                si_v[p, j, pl.ds(q * L, L)] + off)
                        pltpu.async_copy(tbl_hbm.at[gi_v.at[p, j]],
                                         rows_v.at[p, pl.ds(j * 128, 128)],
                                         gsems[p])

            def drain(sem, p):
                pltpu.make_async_copy(tbl_hbm.at[pl.ds(0, SB)],
                                      rows_v.at[p], sem).wait()

            def scatter(p):
                for j in range(S):
                    pltpu.async_copy(rows_v.at[p, pl.ds(j * 128, 128)],
                                     acc_sh.at[di_v.at[p, j]], ssems[p],
                                     add=True)

            for k in range(cpc):
                slot = c if split else c * cpc + k
                off = slot * NPAD if not split else 0
                pltpu.sync_copy(z_hbm.at[pl.ds(s * ROWS_T, ROWS_T)],
                                acc_sh.at[pl.ds(s * ROWS_T, ROWS_T)])
                plsc.subcore_barrier()
                prepare(0, 0, off)
                NB2 = NB // 2

                def body(i, carry):
                    @pl.when(i > 0)
                    def _():
                        drain(ssems[1], 1)

                    prepare(1, 2 * i + 1, off)
                    drain(gsems[0], 0)
                    scatter(0)

                    @pl.when(i + 1 < NB2)
                    def _():
                        drain(ssems[0], 0)
                        prepare(0, 2 * i + 2, off)

                    drain(gsems[1], 1)
                    scatter(1)
                    return carry

                lax.fori_loop(0, NB2, body, 0)
                drain(ssems[0], 0)
                drain(ssems[1], 1)
                plsc.subcore_barrier()
                pltpu.sync_copy(acc_sh.at[pl.ds(s * ROWS_T, ROWS_T)],
                                out_hbm.at[slot, pl.ds(s * ROWS_T, ROWS_T)])
                plsc.subcore_barrier()

        return spmm_k

    spmm1_kernel = make_spmm(D1, NCORE, True, S1)
    spmm2_kernel = make_spmm(L, C2, False, S2)

    xp = jnp.pad(x, ((0, NPAD - N), (0, D1 - IN_DIM)))

    def tc1_body(deg2_ref, xp_ref, dinv_ref, xs1_ref):
        deg = deg2_ref[0, :] + deg2_ref[1, :] + 1.0
        dinv = lax.rsqrt(deg)
        dinv_ref[...] = dinv
        xs1_ref[...] = xp_ref[...] * dinv[:, None]

    R1 = 2048
    dinv, xs1 = pl.pallas_call(
        tc1_body,
        grid=(NPAD // R1,),
        in_specs=[pl.BlockSpec((NCORE, R1), lambda i: (0, i)),
                  pl.BlockSpec((R1, D1), lambda i: (i, 0))],
        out_specs=(pl.BlockSpec((R1,), lambda i: (i,)),
                   pl.BlockSpec((R1, D1), lambda i: (i, 0))),
        out_shape=(jax.ShapeDtypeStruct((NPAD,), F32),
                   jax.ShapeDtypeStruct((NPAD, D1), F32)),
    )(deg2, xp)

    t1 = spmm1_kernel(xs1, srcp, dstp, zeros_d1)

    R = 2048

    def tc2_body(t1_ref, xs1_ref, dinv_ref, W1_ref, b1_ref, out_ref):
        dinv = dinv_ref[...]
        z1 = (t1_ref[0] + t1_ref[1] + xs1_ref[...]) * dinv[:, None]
        h1 = jnp.maximum(
            jnp.dot(z1[:, :IN_DIM], W1_ref[...], preferred_element_type=F32)
            + b1_ref[...][None, :], 0.0)
        xs2 = h1 * dinv[:, None]
        for k in range(C2):
            out_ref[k] = xs2[:, k * L:(k + 1) * L]

    xs2c = pl.pallas_call(
        tc2_body,
        grid=(NPAD // R,),
        in_specs=[pl.BlockSpec((NCORE, R, D1), lambda i: (0, i, 0)),
                  pl.BlockSpec((R, D1), lambda i: (i, 0)),
                  pl.BlockSpec((R,), lambda i: (i,)),
                  pl.BlockSpec((IN_DIM, H), lambda i: (0, 0)),
                  pl.BlockSpec((H,), lambda i: (0,))],
        out_specs=pl.BlockSpec((C2, R, L), lambda i: (0, i, 0)),
        out_shape=jax.ShapeDtypeStruct((C2, NPAD, L), F32),
    )(t1, xs1, dinv, W1, b1)

    t2 = spmm2_kernel(xs2c.reshape(C2 * NPAD, L), srcp, dstp, zeros_d2)

    def tc3_body(t2_ref, xs2_ref, dinv_ref, W2_ref, b2_ref, Wl_ref, bl_ref,
                 out_ref):
        dinv = dinv_ref[...]
        z2 = jnp.concatenate(
            [t2_ref[k] + xs2_ref[k] for k in range(C2)], axis=1) * dinv[:, None]
        h2 = jnp.maximum(
            jnp.dot(z2, W2_ref[...], preferred_element_type=F32)
            + b2_ref[...][None, :], 0.0)
        lg = jnp.dot(h2, Wl_ref[...], preferred_element_type=F32)[:, 0]
        out_ref[...] = lg + bl_ref[0]

    logits_pad = pl.pallas_call(
        tc3_body,
        grid=(NPAD // R,),
        in_specs=[pl.BlockSpec((C2, R, L), lambda i: (0, i, 0)),
                  pl.BlockSpec((C2, R, L), lambda i: (0, i, 0)),
                  pl.BlockSpec((R,), lambda i: (i,)),
                  pl.BlockSpec((H, H), lambda i: (0, 0)),
                  pl.BlockSpec((H,), lambda i: (0,)),
                  pl.BlockSpec((H, 1), lambda i: (0, 0)),
                  pl.BlockSpec((1,), lambda i: (0,))],
        out_specs=pl.BlockSpec((R,), lambda i: (i,)),
        out_shape=jax.ShapeDtypeStruct((NPAD,), F32),
    )(t2, xs2c, dinv, W2, b2, Wl, bl)

    return logits_pad[:N]


# speedup vs baseline: 39.1316x; 1.3849x over previous
import functools

import jax
import jax.numpy as jnp
from jax import lax
from jax.experimental import pallas as pl
from jax.experimental.pallas import tpu as pltpu
from jax.experimental.pallas import tpu_sc as plsc

F32 = jnp.float32
I32 = jnp.int32
L = 16
G = 8
NSUB = 16
NCORE = 2
S = 4


def kernel(x, edge_index, W1, b1, W2, b2, Wl, bl):
    N, IN_DIM = x.shape
    E = edge_index.shape[1]
    H = W1.shape[1]

    NPAD = ((N + 1 + 2047) // 2048) * 2048
    ROWS_T = NPAD // NSUB
    NG = NPAD // G
    EBLK = NCORE * NSUB * 2 * S * 128
    EPAD = ((E + EBLK - 1) // EBLK) * EBLK
    EROWS = EPAD // 128
    RW_SPLIT = EROWS // (NSUB * NCORE)
    RW_FULL = EROWS // NSUB
    NB_SPLIT = RW_SPLIT // S
    NB_FULL = RW_FULL // S
    C2 = H // L
    CPC = C2 // NCORE
    GR_T = ROWS_T // G
    XCH = 8
    GR_CH = GR_T // XCH

    src = edge_index[0].astype(I32)
    dst = edge_index[1].astype(I32)
    pad = EPAD - E
    fill = jnp.arange(pad, dtype=I32)
    srcp = jnp.concatenate([src, fill % 1024]).reshape(EROWS, 128)
    dstp = jnp.concatenate([dst, N + (fill % (NPAD - N))]).reshape(EROWS, 128)

    xg = jnp.pad(x, ((0, NPAD - N), (0, L - IN_DIM))).reshape(NG, 128)
    I8 = jnp.eye(G, dtype=F32)
    W1p = jnp.einsum("ab,jcf->cajbf", I8,
                     jnp.pad(W1, ((0, L - IN_DIM), (0, 0))).reshape(L, C2, L)
                     ).reshape(C2, G * L, G * L)
    b1g = jnp.broadcast_to(b1.reshape(C2, 1, L), (C2, G, L)).reshape(C2, G * L)
    W2p = jnp.einsum("ab,cjdf->cajdbf", I8,
                     W2.reshape(C2, L, C2, L)).reshape(C2 * G * L, C2 * G * L)
    b2c = jnp.broadcast_to(b2.reshape(C2, 1, L), (C2, G, L)).reshape(C2 * G * L)
    Wlp = jnp.einsum("ab,cf->cafb", I8,
                     Wl[:, 0].reshape(C2, L)).reshape(C2 * G * L, G)

    mesh = plsc.VectorSubcoreMesh(core_axis_name="c", subcore_axis_name="s",
                                  num_cores=NCORE, num_subcores=NSUB)
    sc_params = pltpu.CompilerParams(use_tc_tiling_on_sc=False)

    zeros1 = jnp.zeros((NPAD,), F32)
    zeros_d = jnp.zeros((NPAD, L), F32)

    @functools.partial(
        pl.kernel, mesh=mesh,
        out_type=jax.ShapeDtypeStruct((NCORE, NG, 128), F32),
        compiler_params=sc_params,
        scratch_types=[
            pltpu.VMEM((2, S, 128), I32),
            pltpu.VMEM((128,), F32),
            pltpu.VMEM((GR_CH * G,), F32),
            pltpu.VMEM((GR_CH, 128), F32),
            pltpu.VMEM_SHARED((NPAD,), F32),
            pltpu.SemaphoreType.DMA,
            pltpu.SemaphoreType.DMA,
        ],
    )
    def deg_kernel(dst_hbm, z_hbm, out_hbm, di_v, ones_v, dv_v, xp_v, acc_sh,
                   sm0, sm1):
        c = lax.axis_index("c")
        s = lax.axis_index("s")
        sems = (sm0, sm1)
        pltpu.sync_copy(z_hbm.at[pl.ds(s * ROWS_T, ROWS_T)],
                        acc_sh.at[pl.ds(s * ROWS_T, ROWS_T)])
        for i in range(128 // L):
            ones_v[pl.ds(i * L, L)] = jnp.ones((L,), F32)
        plsc.subcore_barrier()
        row0 = c * (NSUB * RW_SPLIT) + s * RW_SPLIT

        def stage(p, b):
            pltpu.sync_copy(dst_hbm.at[pl.ds(row0 + b * S, S)], di_v.at[p])
            for j in range(S):
                pltpu.async_copy(ones_v, acc_sh.at[di_v.at[p, j]], sems[p],
                                 add=True)

        def drain(p):
            pltpu.make_async_copy(dst_hbm.at[pl.ds(0, S)], di_v.at[p],
                                  sems[p]).wait()

        stage(0, 0)
        NB2 = NB_SPLIT // 2

        def body(i, carry):
            @pl.when(i > 0)
            def _():
                drain(1)

            stage(1, 2 * i + 1)

            @pl.when(i + 1 < NB2)
            def _():
                drain(0)
                stage(0, 2 * i + 2)

            return carry

        lax.fori_loop(0, NB2, body, 0)
        drain(0)
        drain(1)
        plsc.subcore_barrier()
        for ch in range(XCH):
            pltpu.sync_copy(
                acc_sh.at[pl.ds(s * ROWS_T + ch * (GR_CH * G), GR_CH * G)],
                dv_v)

            def xbody(r2, carry):
                v16 = dv_v[pl.ds(r2 * (2 * G), 2 * G)]
                for q in range(2 * G):
                    xp_v[2 * r2 + q // G, pl.ds((q % G) * L, L)] = (
                        jnp.full((L,), v16[q], F32))
                return carry

            lax.fori_loop(0, GR_CH // 2, xbody, 0)
            pltpu.sync_copy(
                xp_v, out_hbm.at[c, pl.ds(s * GR_T + ch * GR_CH, GR_CH)])

    deg2g = deg_kernel(dstp, zeros1)

    def make_spmm(n_out, split):
        NB = NB_SPLIT if split else NB_FULL
        cpc = 1 if split else CPC
        SB = S * 128

        @functools.partial(
            pl.kernel, mesh=mesh,
            out_type=jax.ShapeDtypeStruct((n_out, NPAD, L), F32),
            compiler_params=sc_params,
            scratch_types=[
                pltpu.VMEM((2, S, 128), I32),
                pltpu.VMEM((2, S, 128), I32),
                pltpu.VMEM((2, S, 128), I32),
                pltpu.VMEM((2, SB, L), F32),
                pltpu.VMEM_SHARED((NPAD, L), F32),
                pltpu.SemaphoreType.DMA,
                pltpu.SemaphoreType.DMA,
                pltpu.SemaphoreType.DMA,
                pltpu.SemaphoreType.DMA,
            ],
        )
        def spmm_k(tbl_hbm, src_hbm, dst_hbm, z_hbm, out_hbm,
                   si_v, di_v, gi_v, rows_v, acc_sh, gs0, gs1, ss0, ss1):
            c = lax.axis_index("c")
            s = lax.axis_index("s")
            gsems = (gs0, gs1)
            ssems = (ss0, ss1)
            row0 = c * (NSUB * RW_SPLIT) + s * RW_SPLIT if split \
                else s * RW_FULL

            def prepare(p, b, off):
                pltpu.sync_copy(src_hbm.at[pl.ds(row0 + b * S, S)],
                                si_v.at[p])
                pltpu.sync_copy(dst_hbm.at[pl.ds(row0 + b * S, S)],
                                di_v.at[p])
                if split:
                    for j in range(S):
                        pltpu.async_copy(tbl_hbm.at[si_v.at[p, j]],
                                         rows_v.at[p, pl.ds(j * 128, 128)],
                                         gsems[p])
                else:
                    for j in range(S):
                        for q in range(128 // L):
                            gi_v[p, j, pl.ds(q * L, L)] = (
                                si_v[p, j, pl.ds(q * L, L)] + off)
                        pltpu.async_copy(tbl_hbm.at[gi_v.at[p, j]],
                                         rows_v.at[p, pl.ds(j * 128, 128)],
                                         gsems[p])

            def drain(sem, p):
                pltpu.make_async_copy(tbl_hbm.at[pl.ds(0, SB)],
                                      rows_v.at[p], sem).wait()

            def scatter(p):
                for j in range(S):
                    pltpu.async_copy(rows_v.at[p, pl.ds(j * 128, 128)],
                                     acc_sh.at[di_v.at[p, j]], ssems[p],
                                     add=True)

            for k in range(cpc):
                slot = c if split else c * cpc + k
                off = slot * NPAD if not split else 0
                pltpu.sync_copy(z_hbm.at[pl.ds(s * ROWS_T, ROWS_T)],
                                acc_sh.at[pl.ds(s * ROWS_T, ROWS_T)])
                plsc.subcore_barrier()
                prepare(0, 0, off)
                NB2 = NB // 2

                def body(i, carry):
                    @pl.when(i > 0)
                    def _():
                        drain(ssems[1], 1)

                    prepare(1, 2 * i + 1, off)
                    drain(gsems[0], 0)
                    scatter(0)

                    @pl.when(i + 1 < NB2)
                    def _():
                        drain(ssems[0], 0)
                        prepare(0, 2 * i + 2, off)

                    drain(gsems[1], 1)
                    scatter(1)
                    return carry

                lax.fori_loop(0, NB2, body, 0)
                drain(ssems[0], 0)
                drain(ssems[1], 1)
                plsc.subcore_barrier()
                pltpu.sync_copy(acc_sh.at[pl.ds(s * ROWS_T, ROWS_T)],
                                out_hbm.at[slot, pl.ds(s * ROWS_T, ROWS_T)])
                plsc.subcore_barrier()

        return spmm_k

    spmm1_kernel = make_spmm(NCORE, True)
    spmm2_kernel = make_spmm(C2, False)

    RG = 256

    def tc1_body(dg_ref, xg_ref, dinv_ref, xs1_ref):
        dinv = lax.rsqrt(dg_ref[0] + dg_ref[1] + 1.0)
        dinv_ref[...] = dinv
        xs1_ref[...] = xg_ref[...] * dinv

    dinvg, xs1g = pl.pallas_call(
        tc1_body,
        grid=(NG // RG,),
        in_specs=[pl.BlockSpec((NCORE, RG, 128), lambda i: (0, i, 0)),
                  pl.BlockSpec((RG, 128), lambda i: (i, 0))],
        out_specs=(pl.BlockSpec((RG, 128), lambda i: (i, 0)),
                   pl.BlockSpec((RG, 128), lambda i: (i, 0))),
        out_shape=(jax.ShapeDtypeStruct((NG, 128), F32),
                   jax.ShapeDtypeStruct((NG, 128), F32)),
    )(deg2g, xg)

    t1 = spmm1_kernel(xs1g.reshape(NPAD, L), srcp, dstp, zeros_d)
    t1g = t1.reshape(NCORE, NG, 128)

    def tc2_body(t1_ref, xs1_ref, dinv_ref, W1p_ref, b1g_ref, out_ref):
        dinv = dinv_ref[...]
        z1 = (t1_ref[0] + t1_ref[1] + xs1_ref[...]) * dinv
        for k in range(C2):
            h = jnp.dot(z1, W1p_ref[k], preferred_element_type=F32)
            out_ref[k] = jnp.maximum(h + b1g_ref[k][None, :], 0.0) * dinv

    xs2g = pl.pallas_call(
        tc2_body,
        grid=(NG // RG,),
        in_specs=[pl.BlockSpec((NCORE, RG, 128), lambda i: (0, i, 0)),
                  pl.BlockSpec((RG, 128), lambda i: (i, 0)),
                  pl.BlockSpec((RG, 128), lambda i: (i, 0)),
                  pl.BlockSpec((C2, G * L, G * L), lambda i: (0, 0, 0)),
                  pl.BlockSpec((C2, G * L), lambda i: (0, 0))],
        out_specs=pl.BlockSpec((C2, RG, 128), lambda i: (0, i, 0)),
        out_shape=jax.ShapeDtypeStruct((C2, NG, 128), F32),
    )(t1g, xs1g, dinvg, W1p, b1g)

    t2 = spmm2_kernel(xs2g.reshape(C2 * NPAD, L), srcp, dstp, zeros_d)
    t2g = t2.reshape(C2, NG, 128)

    def tc3_body(t2_ref, xs2_ref, dinv_ref, W2p_ref, b2c_ref, Wlp_ref,
                 bl_ref, out_ref):
        dinv = dinv_ref[...]
        zcat = jnp.concatenate(
            [(t2_ref[k] + xs2_ref[k]) * dinv for k in range(C2)], axis=1)
        h2 = jnp.maximum(
            jnp.dot(zcat, W2p_ref[...], preferred_element_type=F32)
            + b2c_ref[...][None, :], 0.0)
        lg = jnp.dot(h2, Wlp_ref[...], preferred_element_type=F32)
        out_ref[...] = lg + bl_ref[0]

    logits_g = pl.pallas_call(
        tc3_body,
        grid=(NG // RG,),
        in_specs=[pl.BlockSpec((C2, RG, 128), lambda i: (0, i, 0)),
                  pl.BlockSpec((C2, RG, 128), lambda i: (0, i, 0)),
                  pl.BlockSpec((RG, 128), lambda i: (i, 0)),
                  pl.BlockSpec((C2 * G * L, C2 * G * L), lambda i: (0, 0)),
                  pl.BlockSpec((C2 * G * L,), lambda i: (0,)),
                  pl.BlockSpec((C2 * G * L, G), lambda i: (0, 0)),
                  pl.BlockSpec((1,), lambda i: (0,))],
        out_specs=pl.BlockSpec((RG, G), lambda i: (i, 0)),
        out_shape=jax.ShapeDtypeStruct((NG, G), F32),
    )(t2g, xs2g, dinvg, W2p, b2c, Wlp, bl)

    return logits_g.reshape(NPAD)[:N]


# speedup vs baseline: 52.9220x; 1.3524x over previous
import functools

import jax
import jax.numpy as jnp
from jax import lax
from jax.experimental import pallas as pl
from jax.experimental.pallas import tpu as pltpu
from jax.experimental.pallas import tpu_sc as plsc

F32 = jnp.float32
I32 = jnp.int32
L = 16
G = 8
NSUB = 16
NCORE = 2
S = 4


def kernel(x, edge_index, W1, b1, W2, b2, Wl, bl):
    N, IN_DIM = x.shape
    E = edge_index.shape[1]
    H = W1.shape[1]

    NPAD = ((N + 1 + 2047) // 2048) * 2048
    ROWS_T = NPAD // NSUB
    NG = NPAD // G
    EBLK = 65536
    EPAD = ((E + EBLK - 1) // EBLK) * EBLK
    EROWS = EPAD // 128
    RW_SPLIT = EROWS // (NSUB * NCORE)
    RW_FULL = EROWS // NSUB
    NB_SPLIT = RW_SPLIT // S
    NB_FULL = RW_FULL // S
    C2 = H // L
    CPC = C2 // NCORE

    EI_ROWS = E // 128
    ei3 = edge_index.astype(I32).reshape(2, EI_ROWS, 128)

    TAIL = EROWS - EI_ROWS

    def tc0_body(ei_ref, srcp_ref, dstp_ref):
        pt = (lax.broadcasted_iota(I32, (TAIL, 128), 0) * 128
              + lax.broadcasted_iota(I32, (TAIL, 128), 1))
        srcp_ref[...] = jnp.concatenate([ei_ref[0], pt % 1024], axis=0)
        dstp_ref[...] = jnp.concatenate([ei_ref[1], N + pt % (NPAD - N)],
                                        axis=0)

    srcp, dstp = pl.pallas_call(
        tc0_body,
        out_shape=(jax.ShapeDtypeStruct((EROWS, 128), I32),
                   jax.ShapeDtypeStruct((EROWS, 128), I32)),
    )(ei3)

    xg = jnp.pad(x, ((0, NPAD - N), (0, L - IN_DIM))).reshape(NG, 128)
    I8 = jnp.eye(G, dtype=F32)
    W1p = jnp.einsum("ab,jcf->cajbf", I8,
                     jnp.pad(W1, ((0, L - IN_DIM), (0, 0))).reshape(L, C2, L)
                     ).reshape(C2, G * L, G * L)
    b1g = jnp.broadcast_to(b1.reshape(C2, 1, L), (C2, G, L)).reshape(C2, G * L)
    W2p = jnp.einsum("ab,cjdf->cajdbf", I8,
                     W2.reshape(C2, L, C2, L)).reshape(C2 * G * L, C2 * G * L)
    b2c = jnp.broadcast_to(b2.reshape(C2, 1, L), (C2, G, L)).reshape(C2 * G * L)
    Wlp = jnp.einsum("ab,cf->cafb", I8,
                     Wl[:, 0].reshape(C2, L)).reshape(C2 * G * L, G)

    mesh = plsc.VectorSubcoreMesh(core_axis_name="c", subcore_axis_name="s",
                                  num_cores=NCORE, num_subcores=NSUB)
    sc_params = pltpu.CompilerParams(use_tc_tiling_on_sc=False)

    zeros1 = jnp.zeros((NPAD,), F32)
    zeros_d = jnp.zeros((NPAD, L), F32)

    @functools.partial(
        pl.kernel, mesh=mesh,
        out_type=jax.ShapeDtypeStruct((NCORE, NPAD), F32),
        compiler_params=sc_params,
        scratch_types=[
            pltpu.VMEM((2, S, 128), I32),
            pltpu.VMEM((128,), F32),
            pltpu.VMEM_SHARED((NPAD,), F32),
            pltpu.SemaphoreType.DMA,
            pltpu.SemaphoreType.DMA,
        ],
    )
    def deg_kernel(dst_hbm, z_hbm, out_hbm, di_v, ones_v, acc_sh, sm0, sm1):
        c = lax.axis_index("c")
        s = lax.axis_index("s")
        sems = (sm0, sm1)
        pltpu.sync_copy(z_hbm.at[pl.ds(s * ROWS_T, ROWS_T)],
                        acc_sh.at[pl.ds(s * ROWS_T, ROWS_T)])
        for i in range(128 // L):
            ones_v[pl.ds(i * L, L)] = jnp.ones((L,), F32)
        plsc.subcore_barrier()
        row0 = c * (NSUB * RW_SPLIT) + s * RW_SPLIT

        def stage(p, b):
            pltpu.sync_copy(dst_hbm.at[pl.ds(row0 + b * S, S)], di_v.at[p])
            for j in range(S):
                pltpu.async_copy(ones_v, acc_sh.at[di_v.at[p, j]], sems[p],
                                 add=True)

        def drain(p):
            pltpu.make_async_copy(dst_hbm.at[pl.ds(0, S)], di_v.at[p],
                                  sems[p]).wait()

        stage(0, 0)
        NB2 = NB_SPLIT // 2

        def body(i, carry):
            @pl.when(i > 0)
            def _():
                drain(1)

            stage(1, 2 * i + 1)

            @pl.when(i + 1 < NB2)
            def _():
                drain(0)
                stage(0, 2 * i + 2)

            return carry

        lax.fori_loop(0, NB2, body, 0)
        drain(0)
        drain(1)
        plsc.subcore_barrier()
        pltpu.sync_copy(acc_sh.at[pl.ds(s * ROWS_T, ROWS_T)],
                        out_hbm.at[c, pl.ds(s * ROWS_T, ROWS_T)])

    deg2 = deg_kernel(dstp, zeros1)
    deg2r = deg2.reshape(NCORE, NPAD // 128, 128)

    def make_spmm(n_out, split):
        NB = NB_SPLIT if split else NB_FULL
        cpc = 1 if split else CPC
        SB = S * 128

        @functools.partial(
            pl.kernel, mesh=mesh,
            out_type=jax.ShapeDtypeStruct((n_out, NPAD, L), F32),
            compiler_params=sc_params,
            scratch_types=[
                pltpu.VMEM((2, S, 128), I32),
                pltpu.VMEM((2, S, 128), I32),
                pltpu.VMEM((2, S, 128), I32),
                pltpu.VMEM((2, SB, L), F32),
                pltpu.VMEM_SHARED((NPAD, L), F32),
                pltpu.SemaphoreType.DMA,
                pltpu.SemaphoreType.DMA,
                pltpu.SemaphoreType.DMA,
                pltpu.SemaphoreType.DMA,
            ],
        )
        def spmm_k(tbl_hbm, src_hbm, dst_hbm, z_hbm, out_hbm,
                   si_v, di_v, gi_v, rows_v, acc_sh, gs0, gs1, ss0, ss1):
            c = lax.axis_index("c")
            s = lax.axis_index("s")
            gsems = (gs0, gs1)
            ssems = (ss0, ss1)
            row0 = c * (NSUB * RW_SPLIT) + s * RW_SPLIT if split \
                else s * RW_FULL

            def prepare(p, b, off):
                pltpu.sync_copy(src_hbm.at[pl.ds(row0 + b * S, S)],
                                si_v.at[p])
                pltpu.sync_copy(dst_hbm.at[pl.ds(row0 + b * S, S)],
                                di_v.at[p])
                if split:
                    for j in range(S):
                        pltpu.async_copy(tbl_hbm.at[si_v.at[p, j]],
                                         rows_v.at[p, pl.ds(j * 128, 128)],
                                         gsems[p])
                else:
                    for j in range(S):
                        for q in range(128 // L):
                            gi_v[p, j, pl.ds(q * L, L)] = (
                                si_v[p, j, pl.ds(q * L, L)] + off)
                        pltpu.async_copy(tbl_hbm.at[gi_v.at[p, j]],
                                         rows_v.at[p, pl.ds(j * 128, 128)],
                                         gsems[p])

            def drain(sem, p):
                pltpu.make_async_copy(tbl_hbm.at[pl.ds(0, SB)],
                                      rows_v.at[p], sem).wait()

            def scatter(p):
                for j in range(S):
                    pltpu.async_copy(rows_v.at[p, pl.ds(j * 128, 128)],
                                     acc_sh.at[di_v.at[p, j]], ssems[p],
                                     add=True)

            for k in range(cpc):
                slot = c if split else c * cpc + k
                off = slot * NPAD if not split else 0
                pltpu.sync_copy(z_hbm.at[pl.ds(s * ROWS_T, ROWS_T)],
                                acc_sh.at[pl.ds(s * ROWS_T, ROWS_T)])
                plsc.subcore_barrier()
                prepare(0, 0, off)
                NB2 = NB // 2

                def body(i, carry):
                    @pl.when(i > 0)
                    def _():
                        drain(ssems[1], 1)

                    prepare(1, 2 * i + 1, off)
                    drain(gsems[0], 0)
                    scatter(0)

                    @pl.when(i + 1 < NB2)
                    def _():
                        drain(ssems[0], 0)
                        prepare(0, 2 * i + 2, off)

                    drain(gsems[1], 1)
                    scatter(1)
                    return carry

                lax.fori_loop(0, NB2, body, 0)
                drain(ssems[0], 0)
                drain(ssems[1], 1)
                plsc.subcore_barrier()
                pltpu.sync_copy(acc_sh.at[pl.ds(s * ROWS_T, ROWS_T)],
                                out_hbm.at[slot, pl.ds(s * ROWS_T, ROWS_T)])
                plsc.subcore_barrier()

        return spmm_k

    spmm1_kernel = make_spmm(NCORE, True)

    SB = S * 128
    SS = 4 * S
    NSB = RW_FULL // SS
    NIT = NSB // 2

    @functools.partial(
        pl.kernel, mesh=mesh,
        out_type=jax.ShapeDtypeStruct((C2, NPAD, L), F32),
        compiler_params=sc_params,
        scratch_types=[
            pltpu.VMEM((2, SS, 128), I32),
            pltpu.VMEM((2, SS, 128), I32),
            pltpu.VMEM((2, SB, L), F32),
            pltpu.VMEM_SHARED((NPAD, L), F32),
            pltpu.SemaphoreType.DMA,
            pltpu.SemaphoreType.DMA,
            pltpu.SemaphoreType.DMA,
            pltpu.SemaphoreType.DMA,
            pltpu.SemaphoreType.DMA,
            pltpu.SemaphoreType.DMA,
        ],
    )
    def spmm2_kernel(tbl_hbm, src_hbm, dst_hbm, z_hbm, out_hbm,
                     si_v, di_v, rows_v, acc_sh, gs0, gs1, ss0, ss1, is0, is1):
        c = lax.axis_index("c")
        s = lax.axis_index("s")
        gsems = (gs0, gs1)
        ssems = (ss0, ss1)
        isems = (is0, is1)
        row0 = s * RW_FULL

        def idx_load(q, sb):
            pltpu.async_copy(src_hbm.at[pl.ds(row0 + sb * SS, SS)],
                             si_v.at[q], isems[q])
            pltpu.async_copy(dst_hbm.at[pl.ds(row0 + sb * SS, SS)],
                             di_v.at[q], isems[q])

        def idx_wait(q):
            pltpu.make_async_copy(src_hbm.at[pl.ds(0, SS)], si_v.at[q],
                                  isems[q]).wait()
            pltpu.make_async_copy(dst_hbm.at[pl.ds(0, SS)], di_v.at[q],
                                  isems[q]).wait()

        def gath(r, q, jb, off):
            for j in range(S):
                for w in range(128 // L):
                    si_v[q, jb + j, pl.ds(w * L, L)] = (
                        si_v[q, jb + j, pl.ds(w * L, L)] + off)
                pltpu.async_copy(tbl_hbm.at[si_v.at[q, jb + j]],
                                 rows_v.at[r, pl.ds(j * 128, 128)], gsems[r])

        def scat(r, q, jb):
            for j in range(S):
                pltpu.async_copy(rows_v.at[r, pl.ds(j * 128, 128)],
                                 acc_sh.at[di_v.at[q, jb + j]], ssems[r],
                                 add=True)

        def gdrain(r):
            pltpu.make_async_copy(tbl_hbm.at[pl.ds(0, SB)], rows_v.at[r],
                                  gsems[r]).wait()

        def sdrain(r):
            pltpu.make_async_copy(tbl_hbm.at[pl.ds(0, SB)], rows_v.at[r],
                                  ssems[r]).wait()

        for k in range(CPC):
            chunk = c * CPC + k
            off = chunk * NPAD
            pltpu.sync_copy(z_hbm.at[pl.ds(s * ROWS_T, ROWS_T)],
                            acc_sh.at[pl.ds(s * ROWS_T, ROWS_T)])
            plsc.subcore_barrier()
            idx_load(0, 0)
            idx_wait(0)
            gath(0, 0, 0, off)

            def body(i, carry):
                @pl.when(i > 0)
                def _():
                    sdrain(1)

                gath(1, 0, S, off)
                gdrain(0)
                scat(0, 0, 0)
                sdrain(0)
                gath(0, 0, 2 * S, off)
                gdrain(1)
                scat(1, 0, S)
                idx_load(1, 2 * i + 1)
                sdrain(1)
                gath(1, 0, 3 * S, off)
                gdrain(0)
                scat(0, 0, 2 * S)
                sdrain(0)
                idx_wait(1)
                gath(0, 1, 0, off)
                gdrain(1)
                scat(1, 0, 3 * S)
                sdrain(1)
                gath(1, 1, S, off)
                gdrain(0)
                scat(0, 1, 0)
                sdrain(0)
                gath(0, 1, 2 * S, off)
                gdrain(1)
                scat(1, 1, S)

                @pl.when(i + 1 < NIT)
                def _():
                    idx_load(0, 2 * i + 2)

                sdrain(1)
                gath(1, 1, 3 * S, off)
                gdrain(0)
                scat(0, 1, 2 * S)

                @pl.when(i + 1 < NIT)
                def _():
                    sdrain(0)
                    idx_wait(0)
                    gath(0, 0, 0, off)

                gdrain(1)
                scat(1, 1, 3 * S)
                return carry

            lax.fori_loop(0, NIT, body, 0)
            sdrain(0)
            sdrain(1)
            plsc.subcore_barrier()
            pltpu.sync_copy(acc_sh.at[pl.ds(s * ROWS_T, ROWS_T)],
                            out_hbm.at[chunk, pl.ds(s * ROWS_T, ROWS_T)])
            plsc.subcore_barrier()

    RG = 256

    RI = RG // L

    def tc1_body(dg_ref, xg_ref, dinv_ref, xs1_ref):
        deg = dg_ref[0] + dg_ref[1] + 1.0
        vexp = jnp.broadcast_to(deg[:, None, :],
                                (RI, L, 128)).reshape(RG, 128)
        rowt = lax.broadcasted_iota(I32, (RG, 128), 0) % L
        lane = lax.broadcasted_iota(I32, (RG, 128), 1)
        degg = jnp.take_along_axis(vexp, rowt * G + lane // L, axis=1)
        dinv = lax.rsqrt(degg)
        dinv_ref[...] = dinv
        xs1_ref[...] = xg_ref[...] * dinv

    dinvg, xs1g = pl.pallas_call(
        tc1_body,
        grid=(NG // RG,),
        in_specs=[pl.BlockSpec((NCORE, RI, 128), lambda i: (0, i, 0)),
                  pl.BlockSpec((RG, 128), lambda i: (i, 0))],
        out_specs=(pl.BlockSpec((RG, 128), lambda i: (i, 0)),
                   pl.BlockSpec((RG, 128), lambda i: (i, 0))),
        out_shape=(jax.ShapeDtypeStruct((NG, 128), F32),
                   jax.ShapeDtypeStruct((NG, 128), F32)),
    )(deg2r, xg)

    t1 = spmm1_kernel(xs1g.reshape(NPAD, L), srcp, dstp, zeros_d)
    t1g = t1.reshape(NCORE, NG, 128)

    def tc2_body(t1_ref, xs1_ref, dinv_ref, W1p_ref, b1g_ref, out_ref):
        dinv = dinv_ref[...]
        z1 = (t1_ref[0] + t1_ref[1] + xs1_ref[...]) * dinv
        for k in range(C2):
            h = jnp.dot(z1, W1p_ref[k], preferred_element_type=F32)
            out_ref[k] = jnp.maximum(h + b1g_ref[k][None, :], 0.0) * dinv

    xs2g = pl.pallas_call(
        tc2_body,
        grid=(NG // RG,),
        in_specs=[pl.BlockSpec((NCORE, RG, 128), lambda i: (0, i, 0)),
                  pl.BlockSpec((RG, 128), lambda i: (i, 0)),
                  pl.BlockSpec((RG, 128), lambda i: (i, 0)),
                  pl.BlockSpec((C2, G * L, G * L), lambda i: (0, 0, 0)),
                  pl.BlockSpec((C2, G * L), lambda i: (0, 0))],
        out_specs=pl.BlockSpec((C2, RG, 128), lambda i: (0, i, 0)),
        out_shape=jax.ShapeDtypeStruct((C2, NG, 128), F32),
    )(t1g, xs1g, dinvg, W1p, b1g)

    t2 = spmm2_kernel(xs2g.reshape(C2 * NPAD, L), srcp, dstp, zeros_d)
    t2g = t2.reshape(C2, NG, 128)

    def tc3_body(t2_ref, xs2_ref, dinv_ref, W2p_ref, b2c_ref, Wlp_ref,
                 bl_ref, out_ref):
        dinv = dinv_ref[...]
        zcat = jnp.concatenate(
            [(t2_ref[k] + xs2_ref[k]) * dinv for k in range(C2)], axis=1)
        h2 = jnp.maximum(
            jnp.dot(zcat, W2p_ref[...], preferred_element_type=F32)
            + b2c_ref[...][None, :], 0.0)
        lg = jnp.dot(h2, Wlp_ref[...], preferred_element_type=F32)
        out_ref[...] = lg + bl_ref[0]

    logits_g = pl.pallas_call(
        tc3_body,
        grid=(NG // RG,),
        in_specs=[pl.BlockSpec((C2, RG, 128), lambda i: (0, i, 0)),
                  pl.BlockSpec((C2, RG, 128), lambda i: (0, i, 0)),
                  pl.BlockSpec((RG, 128), lambda i: (i, 0)),
                  pl.BlockSpec((C2 * G * L, C2 * G * L), lambda i: (0, 0)),
                  pl.BlockSpec((C2 * G * L,), lambda i: (0,)),
                  pl.BlockSpec((C2 * G * L, G), lambda i: (0, 0)),
                  pl.BlockSpec((1,), lambda i: (0,))],
        out_specs=pl.BlockSpec((RG, G), lambda i: (i, 0)),
        out_shape=jax.ShapeDtypeStruct((NG, G), F32),
    )(t2g, xs2g, dinvg, W2p, b2c, Wlp, bl)

    return logits_g.reshape(NPAD)[:N]


# speedup vs baseline: 55.1274x; 1.0417x over previous
import functools

import jax
import jax.numpy as jnp
from jax import lax
from jax.experimental import pallas as pl
from jax.experimental.pallas import tpu as pltpu
from jax.experimental.pallas import tpu_sc as plsc

F32 = jnp.float32
I32 = jnp.int32
L = 16
G = 8
NSUB = 16
NCORE = 2
S = 4


def kernel(x, edge_index, W1, b1, W2, b2, Wl, bl):
    N, IN_DIM = x.shape
    E = edge_index.shape[1]
    H = W1.shape[1]

    NPAD = ((N + 1 + 2047) // 2048) * 2048
    ROWS_T = NPAD // NSUB
    NG = NPAD // G
    EBLK = 65536
    EPAD = ((E + EBLK - 1) // EBLK) * EBLK
    EROWS = EPAD // 128
    RW_SPLIT = EROWS // (NSUB * NCORE)
    RW_FULL = EROWS // NSUB
    NB_SPLIT = RW_SPLIT // S
    SD = 20
    NBD = RW_SPLIT // SD
    NB_FULL = RW_FULL // S
    C2 = H // L
    CPC = C2 // NCORE

    EI_ROWS = E // 128
    ei3 = edge_index.astype(I32).reshape(2, EI_ROWS, 128)

    TAIL = EROWS - EI_ROWS

    def tc0_body(ei_ref, srcp_ref, dstp_ref):
        pt = (lax.broadcasted_iota(I32, (TAIL, 128), 0) * 128
              + lax.broadcasted_iota(I32, (TAIL, 128), 1))
        srcp_ref[...] = jnp.concatenate([ei_ref[0], pt % 1024], axis=0)
        dstp_ref[...] = jnp.concatenate([ei_ref[1], N + pt % (NPAD - N)],
                                        axis=0)

    srcp, dstp = pl.pallas_call(
        tc0_body,
        out_shape=(jax.ShapeDtypeStruct((EROWS, 128), I32),
                   jax.ShapeDtypeStruct((EROWS, 128), I32)),
    )(ei3)

    xc = jnp.pad(x.reshape(N // G, G * IN_DIM), ((0, NG - N // G), (0, 0)))
    I8 = jnp.eye(G, dtype=F32)
    DIL = jnp.einsum("ab,jk->ajbk", I8,
                     jnp.pad(jnp.eye(IN_DIM, dtype=F32),
                             ((0, 0), (0, L - IN_DIM)))
                     ).reshape(G * IN_DIM, G * L)
    W1p = jnp.einsum("ab,jcf->cajbf", I8,
                     jnp.pad(W1, ((0, L - IN_DIM), (0, 0))).reshape(L, C2, L)
                     ).reshape(C2, G * L, G * L)
    b1g = jnp.broadcast_to(b1.reshape(C2, 1, L), (C2, G, L)).reshape(C2, G * L)
    W2p = jnp.einsum("ab,cjdf->cajdbf", I8,
                     W2.reshape(C2, L, C2, L)).reshape(C2 * G * L, C2 * G * L)
    b2c = jnp.broadcast_to(b2.reshape(C2, 1, L), (C2, G, L)).reshape(C2 * G * L)
    Wlp = jnp.einsum("ab,cf->cafb", I8,
                     Wl[:, 0].reshape(C2, L)).reshape(C2 * G * L, G)

    mesh = plsc.VectorSubcoreMesh(core_axis_name="c", subcore_axis_name="s",
                                  num_cores=NCORE, num_subcores=NSUB)
    sc_params = pltpu.CompilerParams(use_tc_tiling_on_sc=False)

    zeros1 = jnp.zeros((NPAD,), F32)
    zeros_d = jnp.zeros((NPAD, L), F32)

    @functools.partial(
        pl.kernel, mesh=mesh,
        out_type=jax.ShapeDtypeStruct((NCORE, NPAD), F32),
        compiler_params=sc_params,
        scratch_types=[
            pltpu.VMEM((2, SD, 128), I32),
            pltpu.VMEM((128,), F32),
            pltpu.VMEM_SHARED((NPAD,), F32),
            pltpu.SemaphoreType.DMA,
            pltpu.SemaphoreType.DMA,
        ],
    )
    def deg_kernel(dst_hbm, z_hbm, out_hbm, di_v, ones_v, acc_sh, sm0, sm1):
        c = lax.axis_index("c")
        s = lax.axis_index("s")
        sems = (sm0, sm1)
        pltpu.sync_copy(z_hbm.at[pl.ds(s * ROWS_T, ROWS_T)],
                        acc_sh.at[pl.ds(s * ROWS_T, ROWS_T)])
        for i in range(128 // L):
            ones_v[pl.ds(i * L, L)] = jnp.ones((L,), F32)
        plsc.subcore_barrier()
        row0 = c * (NSUB * RW_SPLIT) + s * RW_SPLIT

        def stage(p, b):
            pltpu.sync_copy(dst_hbm.at[pl.ds(row0 + b * SD, SD)], di_v.at[p])
            for j in range(SD):
                pltpu.async_copy(ones_v, acc_sh.at[di_v.at[p, j]], sems[p],
                                 add=True)

        def drain(p):
            pltpu.make_async_copy(dst_hbm.at[pl.ds(0, SD)], di_v.at[p],
                                  sems[p]).wait()

        stage(0, 0)
        NB2 = NBD // 2

        def body(i, carry):
            @pl.when(i > 0)
            def _():
                drain(1)

            stage(1, 2 * i + 1)

            @pl.when(i + 1 < NB2)
            def _():
                drain(0)
                stage(0, 2 * i + 2)

            return carry

        lax.fori_loop(0, NB2, body, 0)
        drain(0)
        drain(1)
        plsc.subcore_barrier()
        pltpu.sync_copy(acc_sh.at[pl.ds(s * ROWS_T, ROWS_T)],
                        out_hbm.at[c, pl.ds(s * ROWS_T, ROWS_T)])

    deg2 = deg_kernel(dstp, zeros1)
    deg2r = deg2.reshape(NCORE, NPAD // 128, 128)

    def make_spmm(n_out, split):
        NB = NB_SPLIT if split else NB_FULL
        cpc = 1 if split else CPC
        SB = S * 128

        @functools.partial(
            pl.kernel, mesh=mesh,
            out_type=jax.ShapeDtypeStruct((n_out, NPAD, L), F32),
            compiler_params=sc_params,
            scratch_types=[
                pltpu.VMEM((2, S, 128), I32),
                pltpu.VMEM((2, S, 128), I32),
                pltpu.VMEM((2, S, 128), I32),
                pltpu.VMEM((2, SB, L), F32),
                pltpu.VMEM_SHARED((NPAD, L), F32),
                pltpu.SemaphoreType.DMA,
                pltpu.SemaphoreType.DMA,
                pltpu.SemaphoreType.DMA,
                pltpu.SemaphoreType.DMA,
            ],
        )
        def spmm_k(tbl_hbm, src_hbm, dst_hbm, z_hbm, out_hbm,
                   si_v, di_v, gi_v, rows_v, acc_sh, gs0, gs1, ss0, ss1):
            c = lax.axis_index("c")
            s = lax.axis_index("s")
            gsems = (gs0, gs1)
            ssems = (ss0, ss1)
            row0 = c * (NSUB * RW_SPLIT) + s * RW_SPLIT if split \
                else s * RW_FULL

            def prepare(p, b, off):
                pltpu.sync_copy(src_hbm.at[pl.ds(row0 + b * S, S)],
                                si_v.at[p])
                pltpu.sync_copy(dst_hbm.at[pl.ds(row0 + b * S, S)],
                                di_v.at[p])
                if split:
                    for j in range(S):
                        pltpu.async_copy(tbl_hbm.at[si_v.at[p, j]],
                                         rows_v.at[p, pl.ds(j * 128, 128)],
                                         gsems[p])
                else:
                    for j in range(S):
                        for q in range(128 // L):
                            gi_v[p, j, pl.ds(q * L, L)] = (
                                si_v[p, j, pl.ds(q * L, L)] + off)
                        pltpu.async_copy(tbl_hbm.at[gi_v.at[p, j]],
                                         rows_v.at[p, pl.ds(j * 128, 128)],
                                         gsems[p])

            def drain(sem, p):
                pltpu.make_async_copy(tbl_hbm.at[pl.ds(0, SB)],
                                      rows_v.at[p], sem).wait()

            def scatter(p):
                for j in range(S):
                    pltpu.async_copy(rows_v.at[p, pl.ds(j * 128, 128)],
                                     acc_sh.at[di_v.at[p, j]], ssems[p],
                                     add=True)

            for k in range(cpc):
                slot = c if split else c * cpc + k
                off = slot * NPAD if not split else 0
                pltpu.sync_copy(z_hbm.at[pl.ds(s * ROWS_T, ROWS_T)],
                                acc_sh.at[pl.ds(s * ROWS_T, ROWS_T)])
                plsc.subcore_barrier()
                prepare(0, 0, off)
                NB2 = NB // 2

                def body(i, carry):
                    @pl.when(i > 0)
                    def _():
                        drain(ssems[1], 1)

                    prepare(1, 2 * i + 1, off)
                    drain(gsems[0], 0)
                    scatter(0)

                    @pl.when(i + 1 < NB2)
                    def _():
                        drain(ssems[0], 0)
                        prepare(0, 2 * i + 2, off)

                    drain(gsems[1], 1)
                    scatter(1)
                    return carry

                lax.fori_loop(0, NB2, body, 0)
                drain(ssems[0], 0)
                drain(ssems[1], 1)
                plsc.subcore_barrier()
                pltpu.sync_copy(acc_sh.at[pl.ds(s * ROWS_T, ROWS_T)],
                                out_hbm.at[slot, pl.ds(s * ROWS_T, ROWS_T)])
                plsc.subcore_barrier()

        return spmm_k

    spmm1_kernel = make_spmm(NCORE, True)

    SB = S * 128
    SS = 4 * S
    NSB = RW_FULL // SS
    NIT = NSB // 2

    @functools.partial(
        pl.kernel, mesh=mesh,
        out_type=jax.ShapeDtypeStruct((C2, NPAD, L), F32),
        compiler_params=sc_params,
        scratch_types=[
            pltpu.VMEM((2, SS, 128), I32),
            pltpu.VMEM((2, SS, 128), I32),
            pltpu.VMEM((2, SB, L), F32),
            pltpu.VMEM_SHARED((NPAD, L), F32),
            pltpu.SemaphoreType.DMA,
            pltpu.SemaphoreType.DMA,
            pltpu.SemaphoreType.DMA,
            pltpu.SemaphoreType.DMA,
            pltpu.SemaphoreType.DMA,
            pltpu.SemaphoreType.DMA,
        ],
    )
    def spmm2_kernel(tbl_hbm, src_hbm, dst_hbm, z_hbm, out_hbm,
                     si_v, di_v, rows_v, acc_sh, gs0, gs1, ss0, ss1, is0, is1):
        c = lax.axis_index("c")
        s = lax.axis_index("s")
        gsems = (gs0, gs1)
        ssems = (ss0, ss1)
        isems = (is0, is1)
        row0 = s * RW_FULL

        def idx_load(q, sb):
            pltpu.async_copy(src_hbm.at[pl.ds(row0 + sb * SS, SS)],
                             si_v.at[q], isems[q])
            pltpu.async_copy(dst_hbm.at[pl.ds(row0 + sb * SS, SS)],
                             di_v.at[q], isems[q])

        def idx_wait(q):
            pltpu.make_async_copy(src_hbm.at[pl.ds(0, SS)], si_v.at[q],
                                  isems[q]).wait()
            pltpu.make_async_copy(dst_hbm.at[pl.ds(0, SS)], di_v.at[q],
                                  isems[q]).wait()

        def gath(r, q, jb, off):
            for j in range(S):
                for w in range(128 // L):
                    si_v[q, jb + j, pl.ds(w * L, L)] = (
                        si_v[q, jb + j, pl.ds(w * L, L)] + off)
                pltpu.async_copy(tbl_hbm.at[si_v.at[q, jb + j]],
                                 rows_v.at[r, pl.ds(j * 128, 128)], gsems[r])

        def scat(r, q, jb):
            for j in range(S):
                pltpu.async_copy(rows_v.at[r, pl.ds(j * 128, 128)],
                                 acc_sh.at[di_v.at[q, jb + j]], ssems[r],
                                 add=True)

        def gdrain(r):
            pltpu.make_async_copy(tbl_hbm.at[pl.ds(0, SB)], rows_v.at[r],
                                  gsems[r]).wait()

        def sdrain(r):
            pltpu.make_async_copy(tbl_hbm.at[pl.ds(0, SB)], rows_v.at[r],
                                  ssems[r]).wait()

        for k in range(CPC):
            chunk = c * CPC + k
            off = chunk * NPAD
            pltpu.sync_copy(z_hbm.at[pl.ds(s * ROWS_T, ROWS_T)],
                            acc_sh.at[pl.ds(s * ROWS_T, ROWS_T)])
            plsc.subcore_barrier()
            idx_load(0, 0)
            idx_wait(0)
            gath(0, 0, 0, off)

            def body(i, carry):
                @pl.when(i > 0)
                def _():
                    sdrain(1)

                gath(1, 0, S, off)
                gdrain(0)
                scat(0, 0, 0)
                sdrain(0)
                gath(0, 0, 2 * S, off)
                gdrain(1)
                scat(1, 0, S)
                idx_load(1, 2 * i + 1)
                sdrain(1)
                gath(1, 0, 3 * S, off)
                gdrain(0)
                scat(0, 0, 2 * S)
                sdrain(0)
                idx_wait(1)
                gath(0, 1, 0, off)
                gdrain(1)
                scat(1, 0, 3 * S)
                sdrain(1)
                gath(1, 1, S, off)
                gdrain(0)
                scat(0, 1, 0)
                sdrain(0)
                gath(0, 1, 2 * S, off)
                gdrain(1)
                scat(1, 1, S)

                @pl.when(i + 1 < NIT)
                def _():
                    idx_load(0, 2 * i + 2)

                sdrain(1)
                gath(1, 1, 3 * S, off)
                gdrain(0)
                scat(0, 1, 2 * S)

                @pl.when(i + 1 < NIT)
                def _():
                    sdrain(0)
                    idx_wait(0)
                    gath(0, 0, 0, off)

                gdrain(1)
                scat(1, 1, 3 * S)
                return carry

            lax.fori_loop(0, NIT, body, 0)
            sdrain(0)
            sdrain(1)
            plsc.subcore_barrier()
            pltpu.sync_copy(acc_sh.at[pl.ds(s * ROWS_T, ROWS_T)],
                            out_hbm.at[chunk, pl.ds(s * ROWS_T, ROWS_T)])
            plsc.subcore_barrier()

    RG = 256

    RI = RG // L

    def tc1_body(dg_ref, xc_ref, dil_ref, dinv_ref, xs1_ref):
        deg = dg_ref[0] + dg_ref[1] + 1.0
        vexp = jnp.broadcast_to(deg[:, None, :],
                                (RI, L, 128)).reshape(RG, 128)
        rowt = lax.broadcasted_iota(I32, (RG, 128), 0) % L
        lane = lax.broadcasted_iota(I32, (RG, 128), 1)
        degg = jnp.take_along_axis(vexp, rowt * G + lane // L, axis=1)
        dinv = lax.rsqrt(degg)
        dinv_ref[...] = dinv
        xs1_ref[...] = jnp.dot(xc_ref[...], dil_ref[...],
                               preferred_element_type=F32) * dinv

    dinvg, xs1g = pl.pallas_call(
        tc1_body,
        grid=(NG // RG,),
        in_specs=[pl.BlockSpec((NCORE, RI, 128), lambda i: (0, i, 0)),
                  pl.BlockSpec((RG, G * IN_DIM), lambda i: (i, 0)),
                  pl.BlockSpec((G * IN_DIM, G * L), lambda i: (0, 0))],
        out_specs=(pl.BlockSpec((RG, 128), lambda i: (i, 0)),
                   pl.BlockSpec((RG, 128), lambda i: (i, 0))),
        out_shape=(jax.ShapeDtypeStruct((NG, 128), F32),
                   jax.ShapeDtypeStruct((NG, 128), F32)),
    )(deg2r, xc, DIL)

    t1 = spmm1_kernel(xs1g.reshape(NPAD, L), srcp, dstp, zeros_d)
    t1g = t1.reshape(NCORE, NG, 128)

    def tc2_body(t1_ref, xs1_ref, dinv_ref, W1p_ref, b1g_ref, out_ref):
        dinv = dinv_ref[...]
        z1 = (t1_ref[0] + t1_ref[1] + xs1_ref[...]) * dinv
        for k in range(C2):
            h = jnp.dot(z1, W1p_ref[k], preferred_element_type=F32)
            out_ref[k] = jnp.maximum(h + b1g_ref[k][None, :], 0.0) * dinv

    xs2g = pl.pallas_call(
        tc2_body,
        grid=(NG // RG,),
        in_specs=[pl.BlockSpec((NCORE, RG, 128), lambda i: (0, i, 0)),
                  pl.BlockSpec((RG, 128), lambda i: (i, 0)),
                  pl.BlockSpec((RG, 128), lambda i: (i, 0)),
                  pl.BlockSpec((C2, G * L, G * L), lambda i: (0, 0, 0)),
                  pl.BlockSpec((C2, G * L), lambda i: (0, 0))],
        out_specs=pl.BlockSpec((C2, RG, 128), lambda i: (0, i, 0)),
        out_shape=jax.ShapeDtypeStruct((C2, NG, 128), F32),
    )(t1g, xs1g, dinvg, W1p, b1g)

    t2 = spmm2_kernel(xs2g.reshape(C2 * NPAD, L), srcp, dstp, zeros_d)
    t2g = t2.reshape(C2, NG, 128)

    def tc3_body(t2_ref, xs2_ref, dinv_ref, W2p_ref, b2c_ref, Wlp_ref,
                 bl_ref, out_ref):
        dinv = dinv_ref[...]
        zcat = jnp.concatenate(
            [(t2_ref[k] + xs2_ref[k]) * dinv for k in range(C2)], axis=1)
        h2 = jnp.maximum(
            jnp.dot(zcat, W2p_ref[...], preferred_element_type=F32)
            + b2c_ref[...][None, :], 0.0)
        lg = jnp.dot(h2, Wlp_ref[...], preferred_element_type=F32)
        out_ref[...] = lg + bl_ref[0]

    logits_g = pl.pallas_call(
        tc3_body,
        grid=(NG // RG,),
        in_specs=[pl.BlockSpec((C2, RG, 128), lambda i: (0, i, 0)),
                  pl.BlockSpec((C2, RG, 128), lambda i: (0, i, 0)),
                  pl.BlockSpec((RG, 128), lambda i: (i, 0)),
                  pl.BlockSpec((C2 * G * L, C2 * G * L), lambda i: (0, 0)),
                  pl.BlockSpec((C2 * G * L,), lambda i: (0,)),
                  pl.BlockSpec((C2 * G * L, G), lambda i: (0, 0)),
                  pl.BlockSpec((1,), lambda i: (0,))],
        out_specs=pl.BlockSpec((RG, G), lambda i: (i, 0)),
        out_shape=jax.ShapeDtypeStruct((NG, G), F32),
    )(t2g, xs2g, dinvg, W2p, b2c, Wlp, bl)

    return logits_g.reshape(NPAD)[:N]


# speedup vs baseline: 60.0945x; 1.0901x over previous
import functools

import jax
import jax.numpy as jnp
from jax import lax
from jax.experimental import pallas as pl
from jax.experimental.pallas import tpu as pltpu
from jax.experimental.pallas import tpu_sc as plsc

F32 = jnp.float32
I32 = jnp.int32
L = 16
G = 8
NSUB = 16
NCORE = 2
S = 4


def kernel(x, edge_index, W1, b1, W2, b2, Wl, bl):
    N, IN_DIM = x.shape
    E = edge_index.shape[1]
    H = W1.shape[1]

    NPAD = ((N + 1 + 2047) // 2048) * 2048
    ROWS_T = NPAD // NSUB
    NG = NPAD // G
    EBLK = 65536
    EPAD = ((E + EBLK - 1) // EBLK) * EBLK
    EROWS = EPAD // 128
    RW_SPLIT = EROWS // (NSUB * NCORE)
    RW_FULL = EROWS // NSUB
    NB_SPLIT = RW_SPLIT // S
    SD = 20
    NBD = RW_SPLIT // SD
    NB_FULL = RW_FULL // S
    C2 = H // L
    CPC = C2 // NCORE

    EI_ROWS = E // 128
    ei3 = edge_index.astype(I32).reshape(2, EI_ROWS, 128)

    TAIL = EROWS - EI_ROWS

    def tc0_body(ei_ref, srcp_ref, dstp_ref):
        pt = (lax.broadcasted_iota(I32, (TAIL, 128), 0) * 128
              + lax.broadcasted_iota(I32, (TAIL, 128), 1))
        srcp_ref[...] = jnp.concatenate([ei_ref[0], pt % 1024], axis=0)
        dstp_ref[...] = jnp.concatenate([ei_ref[1], N + pt % (NPAD - N)],
                                        axis=0)

    srcp, dstp = pl.pallas_call(
        tc0_body,
        out_shape=(jax.ShapeDtypeStruct((EROWS, 128), I32),
                   jax.ShapeDtypeStruct((EROWS, 128), I32)),
    )(ei3)

    xc = jnp.pad(x.reshape(N // G, G * IN_DIM), ((0, NG - N // G), (0, 0)))
    I8 = jnp.eye(G, dtype=F32)
    DIL = jnp.einsum("ab,jk->ajbk", I8,
                     jnp.pad(jnp.eye(IN_DIM, dtype=F32),
                             ((0, 0), (0, L - IN_DIM)))
                     ).reshape(G * IN_DIM, G * L)
    W1p = jnp.einsum("ab,jcf->cajbf", I8,
                     jnp.pad(W1, ((0, L - IN_DIM), (0, 0))).reshape(L, C2, L)
                     ).reshape(C2, G * L, G * L)
    b1g = jnp.broadcast_to(b1.reshape(C2, 1, L), (C2, G, L)).reshape(C2, G * L)
    W2p = jnp.einsum("ab,cjdf->cajdbf", I8,
                     W2.reshape(C2, L, C2, L)).reshape(C2 * G * L, C2 * G * L)
    b2c = jnp.broadcast_to(b2.reshape(C2, 1, L), (C2, G, L)).reshape(C2 * G * L)
    Wlp = jnp.einsum("ab,cf->cafb", I8,
                     Wl[:, 0].reshape(C2, L)).reshape(C2 * G * L, G)

    mesh = plsc.VectorSubcoreMesh(core_axis_name="c", subcore_axis_name="s",
                                  num_cores=NCORE, num_subcores=NSUB)
    sc_params = pltpu.CompilerParams(use_tc_tiling_on_sc=False)

    zeros1 = jnp.zeros((NPAD,), F32)
    zeros_d = jnp.zeros((NPAD, L), F32)

    @functools.partial(
        pl.kernel, mesh=mesh,
        out_type=jax.ShapeDtypeStruct((NCORE, NPAD), F32),
        compiler_params=sc_params,
        scratch_types=[
            pltpu.VMEM((2, SD, 128), I32),
            pltpu.VMEM((128,), F32),
            pltpu.VMEM_SHARED((NPAD,), F32),
            pltpu.SemaphoreType.DMA,
            pltpu.SemaphoreType.DMA,
        ],
    )
    def deg_kernel(dst_hbm, z_hbm, out_hbm, di_v, ones_v, acc_sh, sm0, sm1):
        c = lax.axis_index("c")
        s = lax.axis_index("s")
        sems = (sm0, sm1)
        pltpu.sync_copy(z_hbm.at[pl.ds(s * ROWS_T, ROWS_T)],
                        acc_sh.at[pl.ds(s * ROWS_T, ROWS_T)])
        for i in range(128 // L):
            ones_v[pl.ds(i * L, L)] = jnp.ones((L,), F32)
        plsc.subcore_barrier()
        row0 = c * (NSUB * RW_SPLIT) + s * RW_SPLIT

        def stage(p, b):
            pltpu.sync_copy(dst_hbm.at[pl.ds(row0 + b * SD, SD)], di_v.at[p])
            for j in range(SD):
                pltpu.async_copy(ones_v, acc_sh.at[di_v.at[p, j]], sems[p],
                                 add=True)

        def drain(p):
            pltpu.make_async_copy(dst_hbm.at[pl.ds(0, SD)], di_v.at[p],
                                  sems[p]).wait()

        stage(0, 0)
        NB2 = NBD // 2

        def body(i, carry):
            @pl.when(i > 0)
            def _():
                drain(1)

            stage(1, 2 * i + 1)

            @pl.when(i + 1 < NB2)
            def _():
                drain(0)
                stage(0, 2 * i + 2)

            return carry

        lax.fori_loop(0, NB2, body, 0)
        drain(0)
        drain(1)
        plsc.subcore_barrier()
        pltpu.sync_copy(acc_sh.at[pl.ds(s * ROWS_T, ROWS_T)],
                        out_hbm.at[c, pl.ds(s * ROWS_T, ROWS_T)])

    deg2 = deg_kernel(dstp, zeros1)
    deg2r = deg2.reshape(NCORE, NPAD // 128, 128)

    def make_spmm(n_out, split):
        NB = NB_SPLIT if split else NB_FULL
        cpc = 1 if split else CPC
        SB = S * 128

        @functools.partial(
            pl.kernel, mesh=mesh,
            out_type=jax.ShapeDtypeStruct((n_out, NPAD, L), F32),
            compiler_params=sc_params,
            scratch_types=[
                pltpu.VMEM((2, S, 128), I32),
                pltpu.VMEM((2, S, 128), I32),
                pltpu.VMEM((2, S, 128), I32),
                pltpu.VMEM((2, SB, L), F32),
                pltpu.VMEM_SHARED((NPAD, L), F32),
                pltpu.SemaphoreType.DMA,
                pltpu.SemaphoreType.DMA,
                pltpu.SemaphoreType.DMA,
                pltpu.SemaphoreType.DMA,
            ],
        )
        def spmm_k(tbl_hbm, src_hbm, dst_hbm, z_hbm, out_hbm,
                   si_v, di_v, gi_v, rows_v, acc_sh, gs0, gs1, ss0, ss1):
            c = lax.axis_index("c")
            s = lax.axis_index("s")
            gsems = (gs0, gs1)
            ssems = (ss0, ss1)
            row0 = c * (NSUB * RW_SPLIT) + s * RW_SPLIT if split \
                else s * RW_FULL

            def prepare(p, b, off):
                pltpu.sync_copy(src_hbm.at[pl.ds(row0 + b * S, S)],
                                si_v.at[p])
                pltpu.sync_copy(dst_hbm.at[pl.ds(row0 + b * S, S)],
                                di_v.at[p])
                if split:
                    for j in range(S):
                        pltpu.async_copy(tbl_hbm.at[si_v.at[p, j]],
                                         rows_v.at[p, pl.ds(j * 128, 128)],
                                         gsems[p])
                else:
                    for j in range(S):
                        for q in range(128 // L):
                            gi_v[p, j, pl.ds(q * L, L)] = (
                                si_v[p, j, pl.ds(q * L, L)] + off)
                        pltpu.async_copy(tbl_hbm.at[gi_v.at[p, j]],
                                         rows_v.at[p, pl.ds(j * 128, 128)],
                                         gsems[p])

            def drain(sem, p):
                pltpu.make_async_copy(tbl_hbm.at[pl.ds(0, SB)],
                                      rows_v.at[p], sem).wait()

            def scatter(p):
                for j in range(S):
                    pltpu.async_copy(rows_v.at[p, pl.ds(j * 128, 128)],
                                     acc_sh.at[di_v.at[p, j]], ssems[p],
                                     add=True)

            for k in range(cpc):
                slot = c if split else c * cpc + k
                off = slot * NPAD if not split else 0
                pltpu.sync_copy(z_hbm.at[pl.ds(s * ROWS_T, ROWS_T)],
                                acc_sh.at[pl.ds(s * ROWS_T, ROWS_T)])
                plsc.subcore_barrier()
                prepare(0, 0, off)
                NB2 = NB // 2

                def body(i, carry):
                    @pl.when(i > 0)
                    def _():
                        drain(ssems[1], 1)

                    prepare(1, 2 * i + 1, off)
                    drain(gsems[0], 0)
                    scatter(0)

                    @pl.when(i + 1 < NB2)
                    def _():
                        drain(ssems[0], 0)
                        prepare(0, 2 * i + 2, off)

                    drain(gsems[1], 1)
                    scatter(1)
                    return carry

                lax.fori_loop(0, NB2, body, 0)
                drain(ssems[0], 0)
                drain(ssems[1], 1)
                plsc.subcore_barrier()
                pltpu.sync_copy(acc_sh.at[pl.ds(s * ROWS_T, ROWS_T)],
                                out_hbm.at[slot, pl.ds(s * ROWS_T, ROWS_T)])
                plsc.subcore_barrier()

        return spmm_k

    spmm1_kernel = make_spmm(NCORE, True)

    SB = S * 128
    SS = 4 * S
    NSB = RW_FULL // SS
    NIT = NSB // 2

    @functools.partial(
        pl.kernel, mesh=mesh,
        out_type=jax.ShapeDtypeStruct((C2, NPAD, L), F32),
        compiler_params=sc_params,
        scratch_types=[
            pltpu.VMEM((2, SS, 128), I32),
            pltpu.VMEM((2, SS, 128), I32),
            pltpu.VMEM((2, SB, L), F32),
            pltpu.VMEM_SHARED((NPAD, L), F32),
            pltpu.SemaphoreType.DMA,
            pltpu.SemaphoreType.DMA,
            pltpu.SemaphoreType.DMA,
            pltpu.SemaphoreType.DMA,
            pltpu.SemaphoreType.DMA,
            pltpu.SemaphoreType.DMA,
        ],
    )
    def spmm2_kernel(tbl_hbm, src_hbm, dst_hbm, z_hbm, out_hbm,
                     si_v, di_v, rows_v, acc_sh, gs0, gs1, ss0, ss1, is0, is1):
        c = lax.axis_index("c")
        s = lax.axis_index("s")
        gsems = (gs0, gs1)
        ssems = (ss0, ss1)
        isems = (is0, is1)
        row0 = s * RW_FULL

        def idx_load(q, sb):
            pltpu.async_copy(src_hbm.at[pl.ds(row0 + sb * SS, SS)],
                             si_v.at[q], isems[q])
            pltpu.async_copy(dst_hbm.at[pl.ds(row0 + sb * SS, SS)],
                             di_v.at[q], isems[q])

        def idx_wait(q):
            pltpu.make_async_copy(src_hbm.at[pl.ds(0, SS)], si_v.at[q],
                                  isems[q]).wait()
            pltpu.make_async_copy(dst_hbm.at[pl.ds(0, SS)], di_v.at[q],
                                  isems[q]).wait()

        def gath(r, q, jb, off):
            for j in range(S):
                for w in range(128 // L):
                    si_v[q, jb + j, pl.ds(w * L, L)] = (
                        si_v[q, jb + j, pl.ds(w * L, L)] + off)
                pltpu.async_copy(tbl_hbm.at[si_v.at[q, jb + j]],
                                 rows_v.at[r, pl.ds(j * 128, 128)], gsems[r])

        def scat(r, q, jb):
            for j in range(S):
                pltpu.async_copy(rows_v.at[r, pl.ds(j * 128, 128)],
                                 acc_sh.at[di_v.at[q, jb + j]], ssems[r],
                                 add=True)

        def gdrain(r):
            pltpu.make_async_copy(tbl_hbm.at[pl.ds(0, SB)], rows_v.at[r],
                                  gsems[r]).wait()

        def sdrain(r):
            pltpu.make_async_copy(tbl_hbm.at[pl.ds(0, SB)], rows_v.at[r],
                                  ssems[r]).wait()

        for k in range(CPC):
            chunk = c * CPC + k
            off = chunk * NPAD
            pltpu.sync_copy(z_hbm.at[pl.ds(s * ROWS_T, ROWS_T)],
                            acc_sh.at[pl.ds(s * ROWS_T, ROWS_T)])
            plsc.subcore_barrier()
            idx_load(0, 0)
            idx_wait(0)
            gath(0, 0, 0, off)

            def body(i, carry):
                @pl.when(i > 0)
                def _():
                    sdrain(1)

                gath(1, 0, S, off)
                gdrain(0)
                scat(0, 0, 0)
                sdrain(0)
                gath(0, 0, 2 * S, off)
                gdrain(1)
                scat(1, 0, S)
                idx_load(1, 2 * i + 1)
                sdrain(1)
                gath(1, 0, 3 * S, off)
                gdrain(0)
                scat(0, 0, 2 * S)
                sdrain(0)
                idx_wait(1)
                gath(0, 1, 0, off)
                gdrain(1)
                scat(1, 0, 3 * S)
                sdrain(1)
                gath(1, 1, S, off)
                gdrain(0)
                scat(0, 1, 0)
                sdrain(0)
                gath(0, 1, 2 * S, off)
                gdrain(1)
                scat(1, 1, S)

                @pl.when(i + 1 < NIT)
                def _():
                    idx_load(0, 2 * i + 2)

                sdrain(1)
                gath(1, 1, 3 * S, off)
                gdrain(0)
                scat(0, 1, 2 * S)

                @pl.when(i + 1 < NIT)
                def _():
                    sdrain(0)
                    idx_wait(0)
                    gath(0, 0, 0, off)

                gdrain(1)
                scat(1, 1, 3 * S)
                return carry

            lax.fori_loop(0, NIT, body, 0)
            sdrain(0)
            sdrain(1)
            plsc.subcore_barrier()
            pltpu.sync_copy(acc_sh.at[pl.ds(s * ROWS_T, ROWS_T)],
                            out_hbm.at[chunk, pl.ds(s * ROWS_T, ROWS_T)])
            plsc.subcore_barrier()

    RG = 1792

    RI = RG // L

    def tc1_body(dg_ref, xc_ref, dil_ref, dinv_ref, xs1_ref):
        deg = dg_ref[0] + dg_ref[1] + 1.0
        vexp = jnp.broadcast_to(deg[:, None, :],
                                (RI, L, 128)).reshape(RG, 128)
        rowt = lax.broadcasted_iota(I32, (RG, 128), 0) % L
        lane = lax.broadcasted_iota(I32, (RG, 128), 1)
        degg = jnp.take_along_axis(vexp, rowt * G + lane // L, axis=1)
        dinv = lax.rsqrt(degg)
        dinv_ref[...] = dinv
        xs1_ref[...] = jnp.dot(xc_ref[...], dil_ref[...],
                               preferred_element_type=F32) * dinv

    dinvg, xs1g = pl.pallas_call(
        tc1_body,
        grid=(NG // RG,),
        in_specs=[pl.BlockSpec((NCORE, RI, 128), lambda i: (0, i, 0)),
                  pl.BlockSpec((RG, G * IN_DIM), lambda i: (i, 0)),
                  pl.BlockSpec((G * IN_DIM, G * L), lambda i: (0, 0))],
        out_specs=(pl.BlockSpec((RG, 128), lambda i: (i, 0)),
                   pl.BlockSpec((RG, 128), lambda i: (i, 0))),
        out_shape=(jax.ShapeDtypeStruct((NG, 128), F32),
                   jax.ShapeDtypeStruct((NG, 128), F32)),
    )(deg2r, xc, DIL)

    t1 = spmm1_kernel(xs1g.reshape(NPAD, L), srcp, dstp, zeros_d)
    t1g = t1.reshape(NCORE, NG, 128)

    def tc2_body(t1_ref, xs1_ref, dinv_ref, W1p_ref, b1g_ref, out_ref):
        dinv = dinv_ref[...]
        z1 = (t1_ref[0] + t1_ref[1] + xs1_ref[...]) * dinv
        for k in range(C2):
            h = jnp.dot(z1, W1p_ref[k], preferred_element_type=F32)
            out_ref[k] = jnp.maximum(h + b1g_ref[k][None, :], 0.0) * dinv

    xs2g = pl.pallas_call(
        tc2_body,
        grid=(NG // RG,),
        in_specs=[pl.BlockSpec((NCORE, RG, 128), lambda i: (0, i, 0)),
                  pl.BlockSpec((RG, 128), lambda i: (i, 0)),
                  pl.BlockSpec((RG, 128), lambda i: (i, 0)),
                  pl.BlockSpec((C2, G * L, G * L), lambda i: (0, 0, 0)),
                  pl.BlockSpec((C2, G * L), lambda i: (0, 0))],
        out_specs=pl.BlockSpec((C2, RG, 128), lambda i: (0, i, 0)),
        out_shape=jax.ShapeDtypeStruct((C2, NG, 128), F32),
    )(t1g, xs1g, dinvg, W1p, b1g)

    t2 = spmm2_kernel(xs2g.reshape(C2 * NPAD, L), srcp, dstp, zeros_d)
    t2g = t2.reshape(C2, NG, 128)

    def tc3_body(t2_ref, xs2_ref, dinv_ref, W2p_ref, b2c_ref, Wlp_ref,
                 bl_ref, out_ref):
        dinv = dinv_ref[...]
        zcat = jnp.concatenate(
            [(t2_ref[k] + xs2_ref[k]) * dinv for k in range(C2)], axis=1)
        h2 = jnp.maximum(
            jnp.dot(zcat, W2p_ref[...], preferred_element_type=F32)
            + b2c_ref[...][None, :], 0.0)
        lg = jnp.dot(h2, Wlp_ref[...], preferred_element_type=F32)
        out_ref[...] = lg + bl_ref[0]

    logits_g = pl.pallas_call(
        tc3_body,
        grid=(NG // RG,),
        in_specs=[pl.BlockSpec((C2, RG, 128), lambda i: (0, i, 0)),
                  pl.BlockSpec((C2, RG, 128), lambda i: (0, i, 0)),
                  pl.BlockSpec((RG, 128), lambda i: (i, 0)),
                  pl.BlockSpec((C2 * G * L, C2 * G * L), lambda i: (0, 0)),
                  pl.BlockSpec((C2 * G * L,), lambda i: (0,)),
                  pl.BlockSpec((C2 * G * L, G), lambda i: (0, 0)),
                  pl.BlockSpec((1,), lambda i: (0,))],
        out_specs=pl.BlockSpec((RG, G), lambda i: (i, 0)),
        out_shape=jax.ShapeDtypeStruct((NG, G), F32),
    )(t2g, xs2g, dinvg, W2p, b2c, Wlp, bl)

    return logits_g.reshape(NPAD)[:N]
